# baseline (speedup 1.0000x reference)
import functools

import jax
import jax.numpy as jnp
from jax import lax
from jax.experimental import pallas as pl
from jax.experimental.pallas import tpu as pltpu
from jax.experimental.pallas import tpu_sc as plsc

BATCH = 16384
INPUT_WIDTH = 1024
OUTPUT_WIDTH = 2048

NUM_CORES = 2
NUM_SUBCORES = 16
NUM_WORKERS = NUM_CORES * NUM_SUBCORES
ROWS_PER_WORKER = BATCH // NUM_WORKERS
CHUNK_ROWS = 8
NUM_CHUNKS = ROWS_PER_WORKER // CHUNK_ROWS
NBUF = 4
LANES = 16
COL_GROUPS = OUTPUT_WIDTH // LANES


def _sc_body(x_hbm, idx_hbm, out_hbm, idx_v, *bufs):
    x_bufs = bufs[0:NBUF]
    o_bufs = bufs[NBUF:2 * NBUF]
    in_sems = bufs[2 * NBUF:3 * NBUF]
    out_sems = bufs[3 * NBUF:4 * NBUF]

    wid = lax.axis_index("s") * NUM_CORES + lax.axis_index("c")
    base_row = wid * ROWS_PER_WORKER

    def in_slice(chunk):
        return x_hbm.at[pl.ds(base_row + chunk * CHUNK_ROWS, CHUNK_ROWS)]

    def out_slice(chunk):
        return out_hbm.at[pl.ds(base_row + chunk * CHUNK_ROWS, CHUNK_ROWS)]

    def compute(xb, ob):
        @plsc.parallel_loop(0, COL_GROUPS)
        def col_body(jb):
            col = idx_v[pl.ds(jb * LANES, LANES)]
            for r in range(CHUNK_ROWS):
                row_sel = jnp.full((LANES,), r, jnp.int32)
                vals = plsc.load_gather(xb, [row_sel, col])
                ob[r, pl.ds(jb * LANES, LANES)] = vals

    pltpu.sync_copy(idx_hbm, idx_v)

    for b in range(NBUF):
        pltpu.async_copy(in_slice(b), x_bufs[b], in_sems[b])

    for b in range(NBUF):
        pltpu.make_async_copy(in_slice(b), x_bufs[b], in_sems[b]).wait()
        compute(x_bufs[b], o_bufs[b])
        pltpu.async_copy(in_slice(b + NBUF), x_bufs[b], in_sems[b])
        pltpu.async_copy(o_bufs[b], out_slice(b), out_sems[b])

    def loop_body(k, carry):
        for b in range(NBUF):
            chunk = NBUF * k + b
            pltpu.make_async_copy(in_slice(chunk), x_bufs[b], in_sems[b]).wait()
            pltpu.make_async_copy(
                o_bufs[b], out_slice(chunk - NBUF), out_sems[b]
            ).wait()
            compute(x_bufs[b], o_bufs[b])

            @pl.when(chunk + NBUF < NUM_CHUNKS)
            def _prefetch():
                pltpu.async_copy(in_slice(chunk + NBUF), x_bufs[b], in_sems[b])

            pltpu.async_copy(o_bufs[b], out_slice(chunk), out_sems[b])
        return carry

    lax.fori_loop(1, NUM_CHUNKS // NBUF, loop_body, 0)

    for b in range(NBUF):
        pltpu.make_async_copy(
            o_bufs[b], out_slice(NUM_CHUNKS - NBUF + b), out_sems[b]
        ).wait()


def kernel(X, indices):
    mesh = plsc.VectorSubcoreMesh(core_axis_name="c", subcore_axis_name="s")
    f = functools.partial(
        pl.kernel,
        mesh=mesh,
        out_type=jax.ShapeDtypeStruct((BATCH, OUTPUT_WIDTH), jnp.float32),
        compiler_params=pltpu.CompilerParams(
            needs_layout_passes=False,
            use_tc_tiling_on_sc=True,
        ),
        scratch_types=(
            [pltpu.VMEM((OUTPUT_WIDTH,), jnp.int32)]
            + [pltpu.VMEM((CHUNK_ROWS, INPUT_WIDTH), jnp.float32)] * NBUF
            + [pltpu.VMEM((CHUNK_ROWS, OUTPUT_WIDTH), jnp.float32)] * NBUF
            + [pltpu.SemaphoreType.DMA] * (2 * NBUF)
        ),
    )(_sc_body)
    return f(X, indices.astype(jnp.int32))


# speedup vs baseline: 1.0055x; 1.0055x over previous
import functools

import jax
import jax.numpy as jnp
from jax import lax
from jax.experimental import pallas as pl
from jax.experimental.pallas import tpu as pltpu
from jax.experimental.pallas import tpu_sc as plsc

BATCH = 16384
INPUT_WIDTH = 1024
OUTPUT_WIDTH = 2048

NUM_CORES = 2
NUM_SUBCORES = 16
NUM_WORKERS = NUM_CORES * NUM_SUBCORES
ROWS_PER_WORKER = BATCH // NUM_WORKERS
CHUNK_ROWS = 16
NUM_CHUNKS = ROWS_PER_WORKER // CHUNK_ROWS
LANES = 16
COL_GROUPS = OUTPUT_WIDTH // LANES


def _sc_body(
    x_hbm, idx_hbm, out_hbm,
    idx_v, x_v0, x_v1, o_v0, o_v1,
    in_s0, in_s1, out_s0, out_s1,
):
    x_bufs = (x_v0, x_v1)
    o_bufs = (o_v0, o_v1)
    in_sems = (in_s0, in_s1)
    out_sems = (out_s0, out_s1)

    wid = lax.axis_index("s") * NUM_CORES + lax.axis_index("c")
    base_row = wid * ROWS_PER_WORKER

    def in_slice(chunk):
        return x_hbm.at[pl.ds((base_row + chunk * CHUNK_ROWS), CHUNK_ROWS)]

    def out_slice(chunk):
        return out_hbm.at[pl.ds((base_row + chunk * CHUNK_ROWS), CHUNK_ROWS)]

    HALF = CHUNK_ROWS // 2

    def compute_half(xb, ob, half):
        @plsc.parallel_loop(0, COL_GROUPS)
        def col_body(jb):
            col = idx_v[pl.ds(jb * LANES, LANES)]
            for r in range(half * HALF, (half + 1) * HALF):
                row_sel = jnp.full((LANES,), r, jnp.int32)
                vals = plsc.load_gather(xb, [row_sel, col])
                ob[r, pl.ds(jb * LANES, LANES)] = vals

    pltpu.sync_copy(idx_hbm, idx_v)

    pltpu.async_copy(in_slice(0), x_bufs[0], in_sems[0])
    pltpu.async_copy(in_slice(1), x_bufs[1], in_sems[1])

    def out_half_slice(chunk, half):
        row0 = base_row + chunk * CHUNK_ROWS + half * HALF
        return out_hbm.at[pl.ds(row0, HALF)]

    def compute_and_store(chunk, b):
        for half in (0, 1):
            compute_half(x_bufs[b], o_bufs[b], half)
            pltpu.async_copy(
                o_bufs[b].at[pl.ds(half * HALF, HALF)],
                out_half_slice(chunk, half),
                out_sems[b],
            )

    for b in (0, 1):
        pltpu.make_async_copy(in_slice(b), x_bufs[b], in_sems[b]).wait()
        compute_and_store(b, b)
        pltpu.async_copy(in_slice(b + 2), x_bufs[b], in_sems[b])

    def loop_body(k, carry):
        for b in (0, 1):
            chunk = 2 * k + b
            pltpu.make_async_copy(in_slice(chunk), x_bufs[b], in_sems[b]).wait()
            pltpu.make_async_copy(
                o_bufs[b], out_slice(chunk - 2), out_sems[b]
            ).wait()
            compute_and_store(chunk, b)

            @pl.when(chunk + 2 < NUM_CHUNKS)
            def _prefetch():
                pltpu.async_copy(in_slice(chunk + 2), x_bufs[b], in_sems[b])

        return carry

    lax.fori_loop(1, NUM_CHUNKS // 2, loop_body, 0)

    for b in (0, 1):
        pltpu.make_async_copy(
            o_bufs[b], out_slice(NUM_CHUNKS - 2 + b), out_sems[b]
        ).wait()


def kernel(X, indices):
    mesh = plsc.VectorSubcoreMesh(core_axis_name="c", subcore_axis_name="s")
    f = functools.partial(
        pl.kernel,
        mesh=mesh,
        out_type=jax.ShapeDtypeStruct((BATCH, OUTPUT_WIDTH), jnp.float32),
        compiler_params=pltpu.CompilerParams(
            needs_layout_passes=False,
            use_tc_tiling_on_sc=True,
        ),
        scratch_types=[
            pltpu.VMEM((OUTPUT_WIDTH,), jnp.int32),
            pltpu.VMEM((CHUNK_ROWS, INPUT_WIDTH), jnp.float32),
            pltpu.VMEM((CHUNK_ROWS, INPUT_WIDTH), jnp.float32),
            pltpu.VMEM((CHUNK_ROWS, OUTPUT_WIDTH), jnp.float32),
            pltpu.VMEM((CHUNK_ROWS, OUTPUT_WIDTH), jnp.float32),
            pltpu.SemaphoreType.DMA,
            pltpu.SemaphoreType.DMA,
            pltpu.SemaphoreType.DMA,
            pltpu.SemaphoreType.DMA,
        ],
    )(_sc_body)
    return f(X, indices.astype(jnp.int32))
